# baseline (speedup 1.0000x reference)
import functools

import jax
import jax.numpy as jnp
from jax import lax
from jax.experimental import pallas as pl
from jax.experimental.pallas import tpu as pltpu
from jax.experimental.pallas import tpu_sc as plsc

EMB_S = 20
EMB_M = 20
EMB_H = 50
EMB_T = EMB_S + EMB_M + EMB_H
BATCH = 16384
CW = 16
PS = 24
PH = 56

_NC, _NS = 2, 16
_NW = _NC * _NS
_BPW = BATCH // _NW


@functools.cache
def _get_sc_kernel():
  mesh = plsc.VectorSubcoreMesh(core_axis_name="c", subcore_axis_name="s",
                                num_cores=_NC, num_subcores=_NS)

  @functools.partial(
      pl.kernel,
      out_type=jax.ShapeDtypeStruct((BATCH * EMB_T,), jnp.float32),
      mesh=mesh,
      scratch_types=[
          pltpu.VMEM((_BPW,), jnp.int32),
          pltpu.VMEM((_BPW,), jnp.int32),
          pltpu.VMEM((_BPW,), jnp.int32),
          pltpu.VMEM((_BPW, PS), jnp.float32),
          pltpu.VMEM((_BPW, PS), jnp.float32),
          pltpu.VMEM((_BPW, PH), jnp.float32),
          pltpu.VMEM((_BPW * EMB_T + CW,), jnp.float32),
          pltpu.SemaphoreType.DMA,
      ],
      compiler_params=pltpu.CompilerParams(use_tc_tiling_on_sc=False),
  )
  def sc_cat(sidx_hbm, midx_hbm, hidx_hbm, ws_hbm, wm_hbm, wh_hbm, out_hbm,
             si_v, mi_v, hi_v, bs, bm, bh, cat, sem):
    wid = lax.axis_index("s") * _NC + lax.axis_index("c")
    base = wid * _BPW
    pltpu.sync_copy(sidx_hbm.at[pl.ds(base, _BPW)], si_v)
    pltpu.sync_copy(midx_hbm.at[pl.ds(base, _BPW)], mi_v)
    pltpu.sync_copy(hidx_hbm.at[pl.ds(base, _BPW)], hi_v)
    cs = pltpu.async_copy(ws_hbm.at[si_v], bs, sem)
    cm = pltpu.async_copy(wm_hbm.at[mi_v], bm, sem)
    ch = pltpu.async_copy(wh_hbm.at[hi_v], bh, sem)
    cs.wait()
    cm.wait()
    ch.wait()

    plan = ((bs, 0, 0), (bs, 8, 8),
            (bm, 0, EMB_S), (bm, 8, EMB_S + 8),
            (bh, 0, 40), (bh, 16, 56), (bh, 32, 72), (bh, 40, 80))

    def assemble(j2, _):
      for dj in range(2):
        j = 2 * j2 + dj
        rb = EMB_T * j
        for (buf, src, off) in plan:
          v = jnp.reshape(buf[pl.ds(j, 1), pl.ds(src, CW)], (CW,))
          cat[pl.ds(rb + off, CW)] = v
      return 0

    lax.fori_loop(0, _BPW // 2, assemble, 0)
    pltpu.sync_copy(cat.at[pl.ds(0, _BPW * EMB_T)],
                    out_hbm.at[pl.ds(base * EMB_T, _BPW * EMB_T)])

  return sc_cat


def kernel(store_idx, menu_idx, holiday_idx, W_store, W_menu, W_holiday):
  s = store_idx.astype(jnp.int32)
  m = menu_idx.astype(jnp.int32)
  h = holiday_idx.astype(jnp.int32)
  ws_p = jnp.pad(W_store, ((0, 0), (0, PS - EMB_S)))
  wm_p = jnp.pad(W_menu, ((0, 0), (0, PS - EMB_M)))
  wh_p = jnp.pad(W_holiday, ((0, 0), (0, PH - EMB_H)))
  flat = _get_sc_kernel()(s, m, h, ws_p, wm_p, wh_p)
  return flat.reshape(BATCH, EMB_T)


# speedup vs baseline: 1.0288x; 1.0288x over previous
import functools

import jax
import jax.numpy as jnp
from jax import lax
from jax.experimental import pallas as pl
from jax.experimental.pallas import tpu as pltpu
from jax.experimental.pallas import tpu_sc as plsc

EMB_S = 20
EMB_M = 20
EMB_H = 50
EMB_T = EMB_S + EMB_M + EMB_H
BATCH = 16384
CW = 16
PS = 32
PH = 64

_NC, _NS = 2, 16
_NW = _NC * _NS
_BPW = BATCH // _NW


@functools.cache
def _get_sc_kernel():
  mesh = plsc.VectorSubcoreMesh(core_axis_name="c", subcore_axis_name="s",
                                num_cores=_NC, num_subcores=_NS)

  @functools.partial(
      pl.kernel,
      out_type=jax.ShapeDtypeStruct((BATCH * EMB_T,), jnp.float32),
      mesh=mesh,
      scratch_types=[
          pltpu.VMEM((_BPW,), jnp.int32),
          pltpu.VMEM((_BPW,), jnp.int32),
          pltpu.VMEM((_BPW,), jnp.int32),
          pltpu.VMEM((_BPW, PS), jnp.float32),
          pltpu.VMEM((_BPW, PS), jnp.float32),
          pltpu.VMEM((_BPW, PH), jnp.float32),
          pltpu.VMEM((_BPW * EMB_T + CW,), jnp.float32),
          pltpu.SemaphoreType.DMA,
      ],
      compiler_params=pltpu.CompilerParams(use_tc_tiling_on_sc=False),
  )
  def sc_cat(sidx_hbm, midx_hbm, hidx_hbm, ws_hbm, wm_hbm, wh_hbm, out_hbm,
             si_v, mi_v, hi_v, bs, bm, bh, cat, sem):
    wid = lax.axis_index("s") * _NC + lax.axis_index("c")
    base = wid * _BPW
    pltpu.sync_copy(sidx_hbm.at[pl.ds(base, _BPW)], si_v)
    pltpu.sync_copy(midx_hbm.at[pl.ds(base, _BPW)], mi_v)
    pltpu.sync_copy(hidx_hbm.at[pl.ds(base, _BPW)], hi_v)
    cs = pltpu.async_copy(ws_hbm.at[si_v], bs, sem)
    cm = pltpu.async_copy(wm_hbm.at[mi_v], bm, sem)
    ch = pltpu.async_copy(wh_hbm.at[hi_v], bh, sem)
    cs.wait()
    cm.wait()
    ch.wait()

    plan = ((bs, 0, 0), (bs, 16, 16),
            (bm, 0, EMB_S), (bm, 16, EMB_S + 16),
            (bh, 0, 40), (bh, 16, 56), (bh, 32, 72), (bh, 48, 88))

    def assemble(j2, _):
      for dj in range(2):
        j = 2 * j2 + dj
        rb = EMB_T * j
        for (buf, src, off) in plan:
          v = jnp.reshape(buf[pl.ds(j, 1), pl.ds(src, CW)], (CW,))
          cat[pl.ds(rb + off, CW)] = v
      return 0

    lax.fori_loop(0, _BPW // 2, assemble, 0)
    pltpu.sync_copy(cat.at[pl.ds(0, _BPW * EMB_T)],
                    out_hbm.at[pl.ds(base * EMB_T, _BPW * EMB_T)])

  return sc_cat


def kernel(store_idx, menu_idx, holiday_idx, W_store, W_menu, W_holiday):
  s = store_idx.astype(jnp.int32)
  m = menu_idx.astype(jnp.int32)
  h = holiday_idx.astype(jnp.int32)
  ws_p = jnp.pad(W_store, ((0, 0), (0, PS - EMB_S)))
  wm_p = jnp.pad(W_menu, ((0, 0), (0, PS - EMB_M)))
  wh_p = jnp.pad(W_holiday, ((0, 0), (0, PH - EMB_H)))
  flat = _get_sc_kernel()(s, m, h, ws_p, wm_p, wh_p)
  return flat.reshape(BATCH, EMB_T)
